# baseline (device time: 247237 ns/iter reference)
import jax
import jax.numpy as jnp
from jax import lax
from jax.experimental import pallas as pl
from jax.experimental.pallas import tpu as pltpu

N_DEV = 4
N_HOPS = N_DEV - 1

M = 2048
N = 2048
K = 8192
CHUNK = M // N_DEV
BW = 256
N_WT = N // BW
N_KB = 4
BK = K // N_KB
WH = 64
HALF = N // 2


def _chunk_of(ci):
    d = lax.axis_index("i")
    off = jnp.where(ci == 0, 0, jnp.where(ci == 1, 3, jnp.where(ci == 2, 1, 2)))
    return lax.rem(d + off, N_DEV)


def _fused_body(
    dy_ref, w_hbm, acc_ref,
    w_stage, w_bf, dy_bf,
    w_sems,
    comm_cw, comm_ccw,
    send_cw, recv_cw, send_ccw, recv_ccw,
    ag_send_cw, ag_recv_cw, ag_send_ccw, ag_recv_ccw,
):
    ci = pl.program_id(0)
    k = pl.program_id(1)
    w = pl.program_id(2)
    cw_cols = slice(0, HALF)
    ccw_cols = slice(HALF, N)

    d = lax.axis_index("i")
    left = lax.rem(d + N_DEV - 1, N_DEV)
    right = lax.rem(d + 1, N_DEV)

    def row(idx):
        return pl.ds(idx * CHUNK, CHUNK)

    n_q = BW // WH
    n_q_total = N // WH

    def w_sub_load(q, slot):
        return pltpu.make_async_copy(
            w_hbm.at[pl.ds(q * WH, WH), :],
            w_stage.at[slot],
            w_sems.at[slot],
        )

    first_step = jnp.logical_and(ci == 0, jnp.logical_and(k == 0, w == 0))

    @pl.when(first_step)
    def _():
        w_sub_load(0, 0).start()
        w_sub_load(1, 1).start()
        barrier_sem = pltpu.get_barrier_semaphore()
        for nbr in [left, right]:
            pl.semaphore_signal(
                barrier_sem, inc=1,
                device_id=(nbr,), device_id_type=pl.DeviceIdType.MESH,
            )
        pl.semaphore_wait(barrier_sem, 2)

    @pl.when(jnp.logical_and(ci == 0, k == 0))
    def _():
        for j in range(n_q):
            w_sub_load(n_q * w + j, j % 2).wait()
            w_bf[pl.ds(w * BW + j * WH, WH), :] = (
                w_stage[j % 2].astype(jnp.bfloat16)
            )

            nxt = n_q * w + j + 2

            @pl.when(nxt < n_q_total)
            def _():
                w_sub_load(nxt, j % 2).start()

    @pl.when(w == 0)
    def _():
        dy_bf[...] = dy_ref[...].astype(jnp.bfloat16)

    c = _chunk_of(ci)
    tile = lax.dot_general(
        dy_bf[...],
        w_bf[pl.ds(w * BW, BW), pl.ds(k * BK, BK)],
        (((1,), (1,)), ((), ())),
        preferred_element_type=jnp.float32,
    ).astype(jnp.bfloat16)

    @pl.when(k == 0)
    def _():
        acc_ref[row(c), pl.ds(w * BW, BW)] = tile

    @pl.when(k > 0)
    def _():
        acc_ref[row(c), pl.ds(w * BW, BW)] = (
            acc_ref[row(c), pl.ds(w * BW, BW)] + tile
        )

    def rs_hop(h):
        cw_send = lax.rem(d + (N_DEV - h), N_DEV)
        ccw_send = lax.rem(d + h, N_DEV)
        rdma_cw = pltpu.make_async_remote_copy(
            src_ref=acc_ref.at[row(cw_send), cw_cols],
            dst_ref=comm_cw.at[h % 2],
            send_sem=send_cw.at[h],
            recv_sem=recv_cw.at[h],
            device_id=(right,),
            device_id_type=pl.DeviceIdType.MESH,
        )
        rdma_ccw = pltpu.make_async_remote_copy(
            src_ref=acc_ref.at[row(ccw_send), ccw_cols],
            dst_ref=comm_ccw.at[h % 2],
            send_sem=send_ccw.at[h],
            recv_sem=recv_ccw.at[h],
            device_id=(left,),
            device_id_type=pl.DeviceIdType.MESH,
        )
        return rdma_cw, rdma_ccw

    def rs_hop_start(h):
        rdma_cw, rdma_ccw = rs_hop(h)
        rdma_cw.start()
        rdma_ccw.start()

    def rs_hop_finish(h):
        cw_recv = lax.rem(d + (2 * N_DEV - h - 1), N_DEV)
        ccw_recv = lax.rem(d + h + 1, N_DEV)
        rdma_cw, rdma_ccw = rs_hop(h)
        rdma_cw.wait()
        rdma_ccw.wait()
        acc_ref[row(cw_recv), cw_cols] = (
            acc_ref[row(cw_recv), cw_cols] + comm_cw[h % 2]
        )
        acc_ref[row(ccw_recv), ccw_cols] = (
            acc_ref[row(ccw_recv), ccw_cols] + comm_ccw[h % 2]
        )

    chunk_done = jnp.logical_and(k == N_KB - 1, w == N_WT - 1)

    @pl.when(jnp.logical_and(ci == 0, chunk_done))
    def _():
        rs_hop_start(0)

    @pl.when(jnp.logical_and(ci == 2, chunk_done))
    def _():
        rs_hop_finish(0)
        rs_hop_start(1)

    @pl.when(jnp.logical_and(ci == 3, chunk_done))
    def _():
        rs_hop_finish(1)
        rs_hop_start(2)
        rs_hop_finish(2)

        for g in range(N_HOPS):
            cw_send = lax.rem(d + (N_DEV + 1 - g), N_DEV)
            ccw_send = lax.rem(d + (N_DEV - 1 + g), N_DEV)
            rdma_cw = pltpu.make_async_remote_copy(
                src_ref=acc_ref.at[row(cw_send), cw_cols],
                dst_ref=acc_ref.at[row(cw_send), cw_cols],
                send_sem=ag_send_cw.at[g],
                recv_sem=ag_recv_cw.at[g],
                device_id=(right,),
                device_id_type=pl.DeviceIdType.MESH,
            )
            rdma_ccw = pltpu.make_async_remote_copy(
                src_ref=acc_ref.at[row(ccw_send), ccw_cols],
                dst_ref=acc_ref.at[row(ccw_send), ccw_cols],
                send_sem=ag_send_ccw.at[g],
                recv_sem=ag_recv_ccw.at[g],
                device_id=(left,),
                device_id_type=pl.DeviceIdType.MESH,
            )
            rdma_cw.start()
            rdma_ccw.start()
            rdma_cw.wait()
            rdma_ccw.wait()


def _fused_matmul_all_reduce(dy, W):
    sem = pltpu.SemaphoreType.DMA((N_HOPS,))
    return pl.pallas_call(
        _fused_body,
        grid=(N_DEV, N_KB, N_WT),
        in_specs=[
            pl.BlockSpec((CHUNK, BK), lambda ci, k, w: (_chunk_of(ci), k)),
            pl.BlockSpec(memory_space=pl.ANY),
        ],
        out_specs=pl.BlockSpec(memory_space=pltpu.VMEM),
        out_shape=jax.ShapeDtypeStruct((M, N), jnp.bfloat16),
        scratch_shapes=[
            pltpu.VMEM((2, WH, K), jnp.float32),
            pltpu.VMEM((N, K), jnp.bfloat16),
            pltpu.VMEM((CHUNK, BK), jnp.bfloat16),
            pltpu.SemaphoreType.DMA((2,)),
            pltpu.VMEM((2, CHUNK, HALF), jnp.bfloat16),
            pltpu.VMEM((2, CHUNK, HALF), jnp.bfloat16),
            sem, sem, sem, sem,
            sem, sem, sem, sem,
        ],
        compiler_params=pltpu.CompilerParams(
            dimension_semantics=("arbitrary", "arbitrary", "arbitrary"),
            collective_id=0,
            vmem_limit_bytes=63 * 1024 * 1024,
        ),
    )(dy, W)


def kernel(dy, W):
    return _fused_matmul_all_reduce(dy, W).astype(jnp.float32)


# device time: 191383 ns/iter; 1.2918x vs baseline; 1.2918x over previous
import jax
import jax.numpy as jnp
from jax import lax
from jax.experimental import pallas as pl
from jax.experimental.pallas import tpu as pltpu

N_DEV = 4
N_HOPS = N_DEV - 1

M = 2048
N = 2048
K = 8192
CHUNK = M // N_DEV
BW = 256
N_WT = N // BW
HALF = N // 2


def _chunk_of(ci):
    d = lax.axis_index("i")
    off = jnp.where(ci == 0, 0, jnp.where(ci == 1, 3, jnp.where(ci == 2, 1, 2)))
    return lax.rem(d + off, N_DEV)


def _fused_body(
    dy_ref, w_ref, acc_ref,
    comm_cw, comm_ccw,
    send_cw, recv_cw, send_ccw, recv_ccw,
    ag_send_cw, ag_recv_cw, ag_send_ccw, ag_recv_ccw,
):
    ci = pl.program_id(0)
    w = pl.program_id(1)
    cw_cols = slice(0, HALF)
    ccw_cols = slice(HALF, N)

    d = lax.axis_index("i")
    left = lax.rem(d + N_DEV - 1, N_DEV)
    right = lax.rem(d + 1, N_DEV)

    def row(idx):
        return pl.ds(idx * CHUNK, CHUNK)

    @pl.when(jnp.logical_and(ci == 0, w == 0))
    def _():
        barrier_sem = pltpu.get_barrier_semaphore()
        for nbr in [left, right]:
            pl.semaphore_signal(
                barrier_sem, inc=1,
                device_id=(nbr,), device_id_type=pl.DeviceIdType.MESH,
            )
        pl.semaphore_wait(barrier_sem, 2)

    c = _chunk_of(ci)
    tile = lax.dot_general(
        dy_ref[...],
        w_ref[...],
        (((1,), (1,)), ((), ())),
        preferred_element_type=jnp.float32,
    )
    acc_ref[row(c), pl.ds(w * BW, BW)] = tile.astype(jnp.bfloat16)

    def rs_hop(h):
        cw_send = lax.rem(d + (N_DEV - h), N_DEV)
        ccw_send = lax.rem(d + h, N_DEV)
        rdma_cw = pltpu.make_async_remote_copy(
            src_ref=acc_ref.at[row(cw_send), cw_cols],
            dst_ref=comm_cw.at[h],
            send_sem=send_cw.at[h],
            recv_sem=recv_cw.at[h],
            device_id=(right,),
            device_id_type=pl.DeviceIdType.MESH,
        )
        rdma_ccw = pltpu.make_async_remote_copy(
            src_ref=acc_ref.at[row(ccw_send), ccw_cols],
            dst_ref=comm_ccw.at[h],
            send_sem=send_ccw.at[h],
            recv_sem=recv_ccw.at[h],
            device_id=(left,),
            device_id_type=pl.DeviceIdType.MESH,
        )
        return rdma_cw, rdma_ccw

    def rs_hop_start(h):
        rdma_cw, rdma_ccw = rs_hop(h)
        rdma_cw.start()
        rdma_ccw.start()

    def rs_hop_finish(h):
        cw_recv = lax.rem(d + (2 * N_DEV - h - 1), N_DEV)
        ccw_recv = lax.rem(d + h + 1, N_DEV)
        rdma_cw, rdma_ccw = rs_hop(h)
        rdma_cw.wait()
        rdma_ccw.wait()
        acc_ref[row(cw_recv), cw_cols] = (
            acc_ref[row(cw_recv), cw_cols] + comm_cw[h]
        )
        acc_ref[row(ccw_recv), ccw_cols] = (
            acc_ref[row(ccw_recv), ccw_cols] + comm_ccw[h]
        )

    last_w = w == N_WT - 1

    @pl.when(jnp.logical_and(ci == 0, last_w))
    def _():
        rs_hop_start(0)

    @pl.when(jnp.logical_and(ci == 2, last_w))
    def _():
        rs_hop_finish(0)
        rs_hop_start(1)

    @pl.when(jnp.logical_and(ci == 3, last_w))
    def _():
        rs_hop_finish(1)
        rs_hop_start(2)
        rs_hop_finish(2)

        for g in range(N_HOPS):
            cw_send = lax.rem(d + (N_DEV + 1 - g), N_DEV)
            ccw_send = lax.rem(d + (N_DEV - 1 + g), N_DEV)
            rdma_cw = pltpu.make_async_remote_copy(
                src_ref=acc_ref.at[row(cw_send), cw_cols],
                dst_ref=acc_ref.at[row(cw_send), cw_cols],
                send_sem=ag_send_cw.at[g],
                recv_sem=ag_recv_cw.at[g],
                device_id=(right,),
                device_id_type=pl.DeviceIdType.MESH,
            )
            rdma_ccw = pltpu.make_async_remote_copy(
                src_ref=acc_ref.at[row(ccw_send), ccw_cols],
                dst_ref=acc_ref.at[row(ccw_send), ccw_cols],
                send_sem=ag_send_ccw.at[g],
                recv_sem=ag_recv_ccw.at[g],
                device_id=(left,),
                device_id_type=pl.DeviceIdType.MESH,
            )
            rdma_cw.start()
            rdma_ccw.start()
            rdma_cw.wait()
            rdma_ccw.wait()


def _fused_matmul_all_reduce(dy, W):
    sem = pltpu.SemaphoreType.DMA((N_HOPS,))
    return pl.pallas_call(
        _fused_body,
        grid=(N_DEV, N_WT),
        in_specs=[
            pl.BlockSpec((CHUNK, K), lambda ci, w: (_chunk_of(ci), 0)),
            pl.BlockSpec((BW, K), lambda ci, w: (w, 0)),
        ],
        out_specs=pl.BlockSpec(memory_space=pltpu.VMEM),
        out_shape=jax.ShapeDtypeStruct((M, N), jnp.bfloat16),
        scratch_shapes=[
            pltpu.VMEM((N_HOPS, CHUNK, HALF), jnp.bfloat16),
            pltpu.VMEM((N_HOPS, CHUNK, HALF), jnp.bfloat16),
            sem, sem, sem, sem,
            sem, sem, sem, sem,
        ],
        compiler_params=pltpu.CompilerParams(
            dimension_semantics=("arbitrary", "arbitrary"),
            collective_id=0,
            vmem_limit_bytes=62 * 1024 * 1024,
        ),
    )(dy, W)


def kernel(dy, W):
    return _fused_matmul_all_reduce(dy, W).astype(jnp.float32)
